# Initial kernel scaffold; baseline (speedup 1.0000x reference)
import jax
import jax.numpy as jnp
from jax.experimental import pallas as pl


def kernel(X, mask, S_mask):
    raise NotImplementedError("write your pallas kernel here")


# speedup vs baseline: 1.3311x; 1.3311x over previous
import jax
import jax.numpy as jnp
from jax import lax
from jax.experimental import pallas as pl
from jax.experimental.pallas import tpu as pltpu
from jax.experimental.pallas import tpu_sc as plsc

EPS_ = 0.1
M_, D_ = 1000000, 64
TOT = M_ * D_
NC, NS, L = 2, 16, 16
NW = NC * NS
SPAN = TOT // NW
C = 16000
SPW = SPAN // C
NSUB = TOT // C
TCR = 32
SWIN = C + 8
NOWN = (SPW + L - 1) // L


def _counts_body(m_ref, o_ref):
    o_ref[0, 0, :] = jnp.sum(m_ref[...].astype(jnp.int32), axis=1)


def _sc_body(x_hbm, mw_hbm, s_hbm, cnt_hbm, out_hbm,
             cnt_v, off_v, xb, mb, sb, ob):
    wid = lax.axis_index("c") * NS + lax.axis_index("s")

    iota = lax.iota(jnp.int32, L)
    idiv4 = lax.shift_right_logical(iota, 2)
    shv = lax.shift_left(jnp.bitwise_and(iota, 3), 3)

    pltpu.sync_copy(cnt_hbm, cnt_v)

    start = wid * SPW
    nfull = start // L
    rem = start - nfull * L

    def _acc_body(j, acc):
        return acc + cnt_v[pl.ds(j * L, L)]
    acc = lax.fori_loop(0, nfull, _acc_body, jnp.zeros((L,), jnp.int32))
    part = cnt_v[pl.ds(nfull * L, L)]
    acc = acc + jnp.where(iota < rem, part, 0)
    span_off = plsc.cumsum(acc)[L - 1]

    run0 = jnp.full((L,), 0, jnp.int32) + span_off

    def _own_body(j, run):
        c = cnt_v[pl.ds(start + j * L, L)]
        ic = plsc.cumsum(c)
        off_v[pl.ds(j * L, L)] = ic - c + run
        return run + ic[L - 1]
    lax.fori_loop(0, NOWN, _own_body, run0)

    def _sub(k, carry):
        off = off_v[pl.ds(k, L)][0]
        a = pl.multiple_of(jnp.bitwise_and(off, -8), 8)
        sh = off - a
        be = pl.multiple_of(wid * SPAN + k * C, 8)
        bw = pl.multiple_of(wid * (SPAN // 4) + k * (C // 4), 8)
        pltpu.sync_copy(x_hbm.at[pl.ds(be, C)], xb)
        pltpu.sync_copy(mw_hbm.at[pl.ds(bw, C // 4)], mb)
        pltpu.sync_copy(s_hbm.at[pl.ds(a, SWIN)], sb)

        basev0 = jnp.full((L,), 0, jnp.int32) + (sh - 1)

        def _vloop(v, basev):
            widx = jnp.full((L,), 0, jnp.int32) + (4 * v) + idiv4
            w = plsc.load_gather(mb, [widx])
            m = jnp.bitwise_and(lax.shift_right_logical(w, shv), 1)
            mbool = m > 0
            inc = plsc.cumsum(m)
            cnt = plsc.all_reduce_population_count(mbool)
            idx = jnp.maximum(inc + basev, 0)
            g = plsc.load_gather(sb, [idx], mask=mbool)
            x = xb[pl.ds(v * L, L)]
            e = jnp.exp(jnp.clip(x * 2.0, -50.0, 50.0))
            t = (e - 1.0) / (e + 1.0)
            z = jnp.where(mbool, g + EPS_ * t, x)
            ob[pl.ds(v * L, L)] = z
            return basev + cnt

        lax.fori_loop(0, C // L, _vloop, basev0)
        pltpu.sync_copy(ob, out_hbm.at[pl.ds(be, C)])
        return carry

    lax.fori_loop(0, SPW, _sub, 0)


def kernel(X, mask, S_mask):
    x_flat = X.reshape(TOT)
    m_u8 = mask.reshape(TOT).view(jnp.uint8)
    m_words = lax.bitcast_convert_type(m_u8.reshape(TOT // 4, 4), jnp.int32)
    s_pad = jnp.concatenate([S_mask, jnp.zeros((SWIN + 8,), jnp.float32)])

    counts3 = pl.pallas_call(
        _counts_body,
        grid=(NSUB // TCR,),
        in_specs=[pl.BlockSpec((TCR, C), lambda i: (i, 0))],
        out_specs=pl.BlockSpec((1, 1, TCR), lambda i: (i, 0, 0)),
        out_shape=jax.ShapeDtypeStruct((NSUB // TCR, 1, TCR), jnp.int32),
    )(m_u8.reshape(NSUB, C))
    counts = jnp.concatenate(
        [counts3.reshape(NSUB), jnp.zeros((L,), jnp.int32)])

    sc = pl.kernel(
        _sc_body,
        out_type=jax.ShapeDtypeStruct((TOT,), jnp.float32),
        mesh=plsc.VectorSubcoreMesh(core_axis_name="c", subcore_axis_name="s"),
        compiler_params=pltpu.CompilerParams(needs_layout_passes=False),
        scratch_types=[
            pltpu.VMEM((NSUB + L,), jnp.int32),
            pltpu.VMEM((NOWN * L + L,), jnp.int32),
            pltpu.VMEM((C,), jnp.float32),
            pltpu.VMEM((C // 4,), jnp.int32),
            pltpu.VMEM((SWIN,), jnp.float32),
            pltpu.VMEM((C,), jnp.float32),
        ],
    )
    z_flat = sc(x_flat, m_words, s_pad, counts)
    return z_flat.reshape(M_, D_)


# speedup vs baseline: 3.3576x; 2.5225x over previous
import jax
import jax.numpy as jnp
from jax import lax
from jax.experimental import pallas as pl
from jax.experimental.pallas import tpu as pltpu
from jax.experimental.pallas import tpu_sc as plsc

EPS_ = 0.1
M_, D_ = 1000000, 64
NC, NS, L = 2, 16, 16
NW = NC * NS
RPC = 160
C = RPC * D_
NSUB = M_ // RPC
BASE = NSUB // NW
REM = NSUB - BASE * NW
NKMAX = BASE + 1
SWIN = C + 8
NOWN = (NKMAX + L - 1) // L
WWIN = RPC + 8
TCR = 4000


def _counts_body(m_ref, o_cnt, o_w0, o_w1):
    mi = m_ref[...].astype(jnp.int32)
    o_cnt[0, 0, :] = jnp.sum(mi.reshape(TCR // RPC, RPC, D_), axis=(1, 2))
    shifts = jnp.bitwise_and(lax.broadcasted_iota(jnp.int32, (1, D_), 1), 31)
    wp = mi * lax.shift_left(jnp.int32(1), shifts)
    o_w0[0, 0, :] = jnp.sum(wp[:, :32], axis=1)
    o_w1[0, 0, :] = jnp.sum(wp[:, 32:], axis=1)


def _sc_body(x_hbm, w0_hbm, w1_hbm, s_hbm, cnt_hbm, out_hbm,
             cnt_v, off_v, xb0, xb1, wa0, wa1, wb0, wb1, sb0, sb1,
             ob0, ob1, si0, si1, so0, so1):
    wid = lax.axis_index("c") * NS + lax.axis_index("s")
    nk = BASE + jnp.where(wid < REM, 1, 0)
    k0 = wid * BASE + jnp.minimum(wid, REM)

    iota = lax.iota(jnp.int32, L)

    pltpu.sync_copy(cnt_hbm, cnt_v)

    nfull = k0 // L
    rem = k0 - nfull * L

    def _acc_body(j, acc):
        return acc + cnt_v[pl.ds(j * L, L)]
    acc = lax.fori_loop(0, nfull, _acc_body, jnp.zeros((L,), jnp.int32))
    part = cnt_v[pl.ds(nfull * L, L)]
    acc = acc + jnp.where(iota < rem, part, 0)
    span_off = plsc.cumsum(acc)[L - 1]

    run0 = jnp.full((L,), 0, jnp.int32) + span_off

    def _own_body(j, run):
        c = cnt_v[pl.ds(k0 + j * L, L)]
        ic = plsc.cumsum(c)
        off_v[pl.ds(j * L, L)] = ic - c + run
        return run + ic[L - 1]
    lax.fori_loop(0, NOWN, _own_body, run0)

    def in_copies(k, xb, w0b, w1b, sbuf, sem):
        row0 = pl.multiple_of((k0 + k) * RPC, 8)
        off = off_v[pl.ds(k, L)][0]
        a = pl.multiple_of(jnp.bitwise_and(off, -8), 8)
        return [
            pltpu.make_async_copy(x_hbm.at[pl.ds(row0, RPC)], xb, sem),
            pltpu.make_async_copy(w0_hbm.at[pl.ds(row0, WWIN)], w0b, sem),
            pltpu.make_async_copy(w1_hbm.at[pl.ds(row0, WWIN)], w1b, sem),
            pltpu.make_async_copy(s_hbm.at[pl.ds(a, SWIN)], sbuf, sem),
        ]

    def out_copy(k, ob, sem):
        row0 = pl.multiple_of((k0 + k) * RPC, 8)
        return pltpu.make_async_copy(ob, out_hbm.at[pl.ds(row0, RPC)], sem)

    def compute(k, xb, w0b, w1b, sbuf, ob):
        off = off_v[pl.ds(k, L)][0]
        sh = off - jnp.bitwise_and(off, -8)
        basev0 = jnp.full((L,), 0, jnp.int32) + (sh - 1)

        def _row(r, basev):
            widx = jnp.full((L,), 0, jnp.int32) + r
            mw0 = plsc.load_gather(w0b, [widx])
            mw1 = plsc.load_gather(w1b, [widx])
            for q in range(4):
                w = mw0 if q < 2 else mw1
                m = jnp.bitwise_and(
                    lax.shift_right_logical(w, iota + (16 * (q & 1))), 1)
                mbool = m > 0
                inc = plsc.cumsum(m)
                cnt = plsc.all_reduce_population_count(mbool)
                idx = jnp.maximum(inc + basev, 0)
                g = plsc.load_gather(sbuf, [idx], mask=mbool)
                x = xb[r, pl.ds(16 * q, L)]
                e = jnp.exp(jnp.clip(x + x, -50.0, 50.0))
                t = (e - 1.0) / (e + 1.0)
                ob[r, pl.ds(16 * q, L)] = jnp.where(mbool, g + EPS_ * t, x)
                basev = basev + cnt
            return basev

        lax.fori_loop(0, RPC, _row, basev0)

    slots = ((xb0, wa0, wb0, sb0, ob0, si0, so0),
             (xb1, wa1, wb1, sb1, ob1, si1, so1))

    for c in in_copies(0, xb0, wa0, wb0, sb0, si0):
        c.start()

    def _do_k(k, slot, nslot):
        xb, w0b, w1b, sbuf, ob, si, so = slot
        nxb, nw0b, nw1b, nsbuf, _, nsi, _ = nslot

        @pl.when(k >= 2)
        def _():
            out_copy(k - 2, ob, so).wait()

        @pl.when(k + 1 < nk)
        def _():
            for c in in_copies(k + 1, nxb, nw0b, nw1b, nsbuf, nsi):
                c.start()

        for c in in_copies(k, xb, w0b, w1b, sbuf, si):
            c.wait()
        compute(k, xb, w0b, w1b, sbuf, ob)
        out_copy(k, ob, so).start()

    def _step(k, carry):
        pl.when(jnp.logical_and(k < nk, k % 2 == 0))(
            lambda: _do_k(k, slots[0], slots[1]))
        pl.when(jnp.logical_and(k < nk, k % 2 == 1))(
            lambda: _do_k(k, slots[1], slots[0]))
        return carry

    lax.fori_loop(0, NKMAX, _step, 0)

    for d in (2, 1):
        kk = nk - d
        pl.when(kk % 2 == 0)(lambda kk=kk: out_copy(kk, ob0, so0).wait())
        pl.when(kk % 2 == 1)(lambda kk=kk: out_copy(kk, ob1, so1).wait())


def kernel(X, mask, S_mask):
    m_u8 = mask.view(jnp.uint8)
    s_pad = jnp.concatenate([S_mask, jnp.zeros((SWIN + 8,), jnp.float32)])

    counts3, w0, w1 = pl.pallas_call(
        _counts_body,
        grid=(M_ // TCR,),
        in_specs=[pl.BlockSpec((TCR, D_), lambda i: (i, 0))],
        out_specs=[
            pl.BlockSpec((1, 1, TCR // RPC), lambda i: (i, 0, 0)),
            pl.BlockSpec((1, 1, TCR), lambda i: (i, 0, 0)),
            pl.BlockSpec((1, 1, TCR), lambda i: (i, 0, 0)),
        ],
        out_shape=[
            jax.ShapeDtypeStruct((M_ // TCR, 1, TCR // RPC), jnp.int32),
            jax.ShapeDtypeStruct((M_ // TCR, 1, TCR), jnp.int32),
            jax.ShapeDtypeStruct((M_ // TCR, 1, TCR), jnp.int32),
        ],
    )(m_u8)
    w0 = w0.reshape(M_)
    w1 = w1.reshape(M_)
    counts = jnp.concatenate(
        [counts3.reshape(NSUB), jnp.zeros((2 * L,), jnp.int32)])

    sc = pl.kernel(
        _sc_body,
        out_type=jax.ShapeDtypeStruct((M_, D_), jnp.float32),
        mesh=plsc.VectorSubcoreMesh(core_axis_name="c", subcore_axis_name="s"),
        compiler_params=pltpu.CompilerParams(needs_layout_passes=False),
        scratch_types=[
            pltpu.VMEM((NSUB + 2 * L,), jnp.int32),
            pltpu.VMEM((NOWN * L + L,), jnp.int32),
            pltpu.VMEM((RPC, D_), jnp.float32),
            pltpu.VMEM((RPC, D_), jnp.float32),
            pltpu.VMEM((WWIN,), jnp.int32),
            pltpu.VMEM((WWIN,), jnp.int32),
            pltpu.VMEM((WWIN,), jnp.int32),
            pltpu.VMEM((WWIN,), jnp.int32),
            pltpu.VMEM((SWIN,), jnp.float32),
            pltpu.VMEM((SWIN,), jnp.float32),
            pltpu.VMEM((RPC, D_), jnp.float32),
            pltpu.VMEM((RPC, D_), jnp.float32),
            pltpu.SemaphoreType.DMA,
            pltpu.SemaphoreType.DMA,
            pltpu.SemaphoreType.DMA,
            pltpu.SemaphoreType.DMA,
        ],
    )
    return sc(X, w0, w1, s_pad, counts)
